# baseline (speedup 1.0000x reference)
import functools

import jax
import jax.numpy as jnp
from jax import lax
from jax.experimental import pallas as pl
from jax.experimental.pallas import tpu as pltpu
from jax.experimental.pallas import tpu_sc as plsc

N_TOK = 32768
D_MODEL = 4096
N_EXP = 64
K_TOP = 8
BLK = 1024
N_CHUNKS = 1
LANES = 16
N_WORKERS = 32


def _gate_body(h_ref, w_ref, logit_ref):
    lt = jax.lax.dot_general(
        w_ref[...], h_ref[...], (((1,), (1,)), ((), ())),
        preferred_element_type=jnp.float32)
    logit_ref[...] = lt.T


def _make_gate(chunk_tokens, chunk_idx):
    nblk = chunk_tokens // BLK
    base_blk = chunk_idx * nblk
    return pl.pallas_call(
        _gate_body,
        grid=(nblk,),
        in_specs=[
            pl.BlockSpec((BLK, D_MODEL), lambda i: (base_blk + i, 0)),
            pl.BlockSpec((N_EXP, D_MODEL), lambda i: (0, 0)),
        ],
        out_specs=pl.BlockSpec((BLK, N_EXP), lambda i: (i, 0)),
        out_shape=jax.ShapeDtypeStruct((chunk_tokens, N_EXP), jnp.float32),
    )


def _sc_route_body(tpw, logit_hbm, idx_hbm, wgt_hbm, lg_v, oi_v, ow_v):
    wid = lax.axis_index("s") * 2 + lax.axis_index("c")
    base = wid * tpw
    pltpu.sync_copy(logit_hbm.at[pl.ds(base * N_EXP, tpw * N_EXP)], lg_v)

    lane = lax.iota(jnp.int32, LANES)
    lane_e = lane * N_EXP
    lane_k = lane * K_TOP

    def group(g, _):
        gbase_e = g * (LANES * N_EXP)
        gbase_k = g * (LANES * K_TOP)
        neg_inf = jnp.full((LANES,), -jnp.inf, jnp.float32)
        s = [neg_inf for _ in range(K_TOP)]
        si = [jnp.zeros((LANES,), jnp.int32) for _ in range(K_TOP)]
        m = neg_inf
        for e in range(N_EXP):
            x = plsc.load_gather(lg_v, [lane_e + (gbase_e + e)])
            m = jnp.maximum(m, x)
            xi = jnp.full((LANES,), e, jnp.int32)
            for j in range(K_TOP):
                c = x > s[j]
                nv = jnp.where(c, x, s[j])
                ni = jnp.where(c, xi, si[j])
                x = jnp.where(c, s[j], x)
                xi = jnp.where(c, si[j], xi)
                s[j], si[j] = nv, ni
        acc = jnp.zeros((LANES,), jnp.float32)
        for e in range(N_EXP):
            x = plsc.load_gather(lg_v, [lane_e + (gbase_e + e)])
            acc = acc + jnp.exp(x - m)
        for j in range(K_TOP):
            oidx = lane_k + (gbase_k + j)
            plsc.store_scatter(oi_v, [oidx], si[j])
            plsc.store_scatter(ow_v, [oidx], jnp.exp(s[j] - m) / acc)
        return _

    lax.fori_loop(0, tpw // LANES, group, None)

    pltpu.sync_copy(oi_v, idx_hbm.at[pl.ds(base * K_TOP, tpw * K_TOP)])
    pltpu.sync_copy(ow_v, wgt_hbm.at[pl.ds(base * K_TOP, tpw * K_TOP)])


def _make_sc_route(chunk_tokens):
    tpw = chunk_tokens // N_WORKERS
    mesh = plsc.VectorSubcoreMesh(core_axis_name="c", subcore_axis_name="s")
    return pl.kernel(
        functools.partial(_sc_route_body, tpw),
        mesh=mesh,
        out_type=[
            jax.ShapeDtypeStruct((chunk_tokens * K_TOP,), jnp.int32),
            jax.ShapeDtypeStruct((chunk_tokens * K_TOP,), jnp.float32),
        ],
        scratch_types=[
            pltpu.VMEM((tpw * N_EXP,), jnp.float32),
            pltpu.VMEM((tpw * K_TOP,), jnp.int32),
            pltpu.VMEM((tpw * K_TOP,), jnp.float32),
        ],
        compiler_params=pltpu.CompilerParams(needs_layout_passes=False),
    )


def kernel(layer_idx, hidden, W):
    n_tok = hidden.shape[0]
    ct = n_tok // N_CHUNKS
    sc_route = _make_sc_route(ct)
    idx_c, wgt_c, logit_c = [], [], []
    for c in range(N_CHUNKS):
        logits = _make_gate(ct, c)(hidden, W)
        idx, wgt = sc_route(logits.reshape(ct * N_EXP))
        idx_c.append(idx.reshape(ct, K_TOP))
        wgt_c.append(wgt.reshape(ct, K_TOP))
        logit_c.append(logits)
    if N_CHUNKS == 1:
        return (idx_c[0], wgt_c[0], logit_c[0])
    return (jnp.concatenate(idx_c, 0), jnp.concatenate(wgt_c, 0),
            jnp.concatenate(logit_c, 0))


# speedup vs baseline: 1.0154x; 1.0154x over previous
import functools

import jax
import jax.numpy as jnp
from jax import lax
from jax.experimental import pallas as pl
from jax.experimental.pallas import tpu as pltpu
from jax.experimental.pallas import tpu_sc as plsc

N_TOK = 32768
D_MODEL = 4096
N_EXP = 64
K_TOP = 8
BLK = 1024
N_CHUNKS = 8
LANES = 16
N_WORKERS = 32


def _gate_body(h_ref, w_ref, logit_ref):
    lt = jax.lax.dot_general(
        w_ref[...], h_ref[...], (((1,), (1,)), ((), ())),
        preferred_element_type=jnp.float32)
    logit_ref[...] = lt.T


def _make_gate(chunk_tokens, chunk_idx):
    nblk = chunk_tokens // BLK
    base_blk = chunk_idx * nblk
    return pl.pallas_call(
        _gate_body,
        grid=(nblk,),
        in_specs=[
            pl.BlockSpec((BLK, D_MODEL), lambda i: (base_blk + i, 0)),
            pl.BlockSpec((N_EXP, D_MODEL), lambda i: (0, 0)),
        ],
        out_specs=pl.BlockSpec((BLK, N_EXP), lambda i: (i, 0)),
        out_shape=jax.ShapeDtypeStruct((chunk_tokens, N_EXP), jnp.float32),
    )


def _sc_route_body(tpw, logit_hbm, idx_hbm, wgt_hbm, lg_v, oi_v, ow_v):
    wid = lax.axis_index("s") * 2 + lax.axis_index("c")
    base = wid * tpw
    pltpu.sync_copy(logit_hbm.at[pl.ds(base * N_EXP, tpw * N_EXP)], lg_v)

    lane = lax.iota(jnp.int32, LANES)
    lane_e = lane * N_EXP
    lane_k = lane * K_TOP

    def group(g, _):
        gbase_e = g * (LANES * N_EXP)
        gbase_k = g * (LANES * K_TOP)
        neg_inf = jnp.full((LANES,), -jnp.inf, jnp.float32)
        s = [neg_inf for _ in range(K_TOP)]
        si = [jnp.zeros((LANES,), jnp.int32) for _ in range(K_TOP)]
        m = neg_inf
        for e in range(N_EXP):
            x = plsc.load_gather(lg_v, [lane_e + (gbase_e + e)])
            m = jnp.maximum(m, x)
            xi = jnp.full((LANES,), e, jnp.int32)
            for j in range(K_TOP):
                c = x > s[j]
                nv = jnp.where(c, x, s[j])
                ni = jnp.where(c, xi, si[j])
                x = jnp.where(c, s[j], x)
                xi = jnp.where(c, si[j], xi)
                s[j], si[j] = nv, ni
        acc = jnp.zeros((LANES,), jnp.float32)
        for e in range(N_EXP):
            x = plsc.load_gather(lg_v, [lane_e + (gbase_e + e)])
            acc = acc + jnp.exp(x - m)
        for j in range(K_TOP):
            oidx = lane_k + (gbase_k + j)
            plsc.store_scatter(oi_v, [oidx], si[j])
            plsc.store_scatter(ow_v, [oidx], jnp.exp(s[j] - m) / acc)
        return _

    lax.fori_loop(0, tpw // LANES, group, None)

    pltpu.sync_copy(oi_v, idx_hbm.at[pl.ds(base * K_TOP, tpw * K_TOP)])
    pltpu.sync_copy(ow_v, wgt_hbm.at[pl.ds(base * K_TOP, tpw * K_TOP)])


def _make_sc_route(chunk_tokens):
    tpw = chunk_tokens // N_WORKERS
    mesh = plsc.VectorSubcoreMesh(core_axis_name="c", subcore_axis_name="s")
    return pl.kernel(
        functools.partial(_sc_route_body, tpw),
        mesh=mesh,
        out_type=[
            jax.ShapeDtypeStruct((chunk_tokens * K_TOP,), jnp.int32),
            jax.ShapeDtypeStruct((chunk_tokens * K_TOP,), jnp.float32),
        ],
        scratch_types=[
            pltpu.VMEM((tpw * N_EXP,), jnp.float32),
            pltpu.VMEM((tpw * K_TOP,), jnp.int32),
            pltpu.VMEM((tpw * K_TOP,), jnp.float32),
        ],
        compiler_params=pltpu.CompilerParams(needs_layout_passes=False),
    )


def kernel(layer_idx, hidden, W):
    n_tok = hidden.shape[0]
    ct = n_tok // N_CHUNKS
    sc_route = _make_sc_route(ct)
    idx_c, wgt_c, logit_c = [], [], []
    for c in range(N_CHUNKS):
        logits = _make_gate(ct, c)(hidden, W)
        idx, wgt = sc_route(logits.reshape(ct * N_EXP))
        idx_c.append(idx.reshape(ct, K_TOP))
        wgt_c.append(wgt.reshape(ct, K_TOP))
        logit_c.append(logits)
    if N_CHUNKS == 1:
        return (idx_c[0], wgt_c[0], logit_c[0])
    return (jnp.concatenate(idx_c, 0), jnp.concatenate(wgt_c, 0),
            jnp.concatenate(logit_c, 0))


# speedup vs baseline: 1.1757x; 1.1579x over previous
import functools

import jax
import jax.numpy as jnp
from jax import lax
from jax.experimental import pallas as pl
from jax.experimental.pallas import tpu as pltpu
from jax.experimental.pallas import tpu_sc as plsc

N_TOK = 32768
D_MODEL = 4096
N_EXP = 64
K_TOP = 8
BLK = 1024
LANES = 16
N_WORKERS = 32
SC_TOKENS = 4096


def _router_body(h_ref, w_ref, idx_ref, wgt_ref, logit_ref):
    lt = jax.lax.dot_general(
        w_ref[...], h_ref[...], (((1,), (1,)), ((), ())),
        preferred_element_type=jnp.float32)
    logit_ref[...] = lt.T

    m = jnp.max(lt, axis=0, keepdims=True)
    ex = jnp.exp(lt - m)
    probs = ex / jnp.sum(ex, axis=0, keepdims=True)

    cur = probs
    e_iota = jax.lax.broadcasted_iota(jnp.int32, cur.shape, 0)
    idx_rows = []
    wgt_rows = []
    for _ in range(K_TOP):
        mx = jnp.max(cur, axis=0, keepdims=True)
        amax = jnp.min(jnp.where(cur == mx, e_iota, N_EXP),
                       axis=0, keepdims=True)
        idx_rows.append(amax)
        wgt_rows.append(mx)
        cur = jnp.where(e_iota == amax, -1.0, cur)
    idx_ref[...] = jnp.concatenate(idx_rows, axis=0).T
    wgt_ref[...] = jnp.concatenate(wgt_rows, axis=0).T


def _tc_router(hidden, W):
    n_tok = hidden.shape[0]
    return pl.pallas_call(
        _router_body,
        grid=(n_tok // BLK,),
        in_specs=[
            pl.BlockSpec((BLK, D_MODEL), lambda i: (i, 0)),
            pl.BlockSpec((N_EXP, D_MODEL), lambda i: (0, 0)),
        ],
        out_specs=(
            pl.BlockSpec((BLK, K_TOP), lambda i: (i, 0)),
            pl.BlockSpec((BLK, K_TOP), lambda i: (i, 0)),
            pl.BlockSpec((BLK, N_EXP), lambda i: (i, 0)),
        ),
        out_shape=(
            jax.ShapeDtypeStruct((n_tok, K_TOP), jnp.int32),
            jax.ShapeDtypeStruct((n_tok, K_TOP), jnp.float32),
            jax.ShapeDtypeStruct((n_tok, N_EXP), jnp.float32),
        ),
    )(hidden, W)


def _sc_route_body(tpw, logit_hbm, idx_hbm, wgt_hbm, lg_v, oi_v, ow_v):
    wid = lax.axis_index("s") * 2 + lax.axis_index("c")
    base = wid * tpw
    pltpu.sync_copy(logit_hbm.at[pl.ds(base * N_EXP, tpw * N_EXP)], lg_v)

    lane = lax.iota(jnp.int32, LANES)
    lane_e = lane * N_EXP
    lane_k = lane * K_TOP

    def group(g, _):
        gbase_e = g * (LANES * N_EXP)
        gbase_k = g * (LANES * K_TOP)
        neg_inf = jnp.full((LANES,), -jnp.inf, jnp.float32)
        s = [neg_inf for _ in range(K_TOP)]
        si = [jnp.zeros((LANES,), jnp.int32) for _ in range(K_TOP)]
        m = neg_inf
        for e in range(N_EXP):
            x = plsc.load_gather(lg_v, [lane_e + (gbase_e + e)])
            m = jnp.maximum(m, x)
            xi = jnp.full((LANES,), e, jnp.int32)
            for j in range(K_TOP):
                c = x > s[j]
                nv = jnp.where(c, x, s[j])
                ni = jnp.where(c, xi, si[j])
                x = jnp.where(c, s[j], x)
                xi = jnp.where(c, si[j], xi)
                s[j], si[j] = nv, ni
        acc = jnp.zeros((LANES,), jnp.float32)
        for e in range(N_EXP):
            x = plsc.load_gather(lg_v, [lane_e + (gbase_e + e)])
            acc = acc + jnp.exp(x - m)
        for j in range(K_TOP):
            oidx = lane_k + (gbase_k + j)
            plsc.store_scatter(oi_v, [oidx], si[j])
            plsc.store_scatter(ow_v, [oidx], jnp.exp(s[j] - m) / acc)
        return _

    lax.fori_loop(0, tpw // LANES, group, None)

    pltpu.sync_copy(oi_v, idx_hbm.at[pl.ds(base * K_TOP, tpw * K_TOP)])
    pltpu.sync_copy(ow_v, wgt_hbm.at[pl.ds(base * K_TOP, tpw * K_TOP)])


def _make_sc_route(sc_tokens):
    tpw = sc_tokens // N_WORKERS
    mesh = plsc.VectorSubcoreMesh(core_axis_name="c", subcore_axis_name="s")
    return pl.kernel(
        functools.partial(_sc_route_body, tpw),
        mesh=mesh,
        out_type=[
            jax.ShapeDtypeStruct((sc_tokens * K_TOP,), jnp.int32),
            jax.ShapeDtypeStruct((sc_tokens * K_TOP,), jnp.float32),
        ],
        scratch_types=[
            pltpu.VMEM((tpw * N_EXP,), jnp.float32),
            pltpu.VMEM((tpw * K_TOP,), jnp.int32),
            pltpu.VMEM((tpw * K_TOP,), jnp.float32),
        ],
        compiler_params=pltpu.CompilerParams(needs_layout_passes=False),
    )


def kernel(layer_idx, hidden, W):
    idx_tc, wgt_tc, logits = _tc_router(hidden, W)
    sc_idx, sc_wgt = _make_sc_route(SC_TOKENS)(logits.reshape(-1))
    idx = jnp.concatenate(
        [sc_idx.reshape(SC_TOKENS, K_TOP), idx_tc[SC_TOKENS:]], axis=0)
    wgt = jnp.concatenate(
        [sc_wgt.reshape(SC_TOKENS, K_TOP), wgt_tc[SC_TOKENS:]], axis=0)
    return (idx, wgt, logits)


# speedup vs baseline: 1.2814x; 1.0899x over previous
import functools

import jax
import jax.numpy as jnp
from jax import lax
from jax.experimental import pallas as pl
from jax.experimental.pallas import tpu as pltpu
from jax.experimental.pallas import tpu_sc as plsc

N_TOK = 32768
D_MODEL = 4096
N_EXP = 64
K_TOP = 8
BLK = 1024
LANES = 16
N_WORKERS = 32
SC_TOKENS = 4096


def _router_body(h_ref, w_ref, idx_ref, wgt_ref, logit_ref):
    lt = jax.lax.dot_general(
        w_ref[...], h_ref[...], (((1,), (1,)), ((), ())),
        preferred_element_type=jnp.float32)
    logit_ref[...] = lt.T

    m = jnp.max(lt, axis=0, keepdims=True)
    ex = jnp.exp(lt - m)
    probs = ex / jnp.sum(ex, axis=0, keepdims=True)

    cur = probs
    e_iota = jax.lax.broadcasted_iota(jnp.int32, cur.shape, 0)
    idx_rows = []
    wgt_rows = []
    for _ in range(K_TOP):
        mx = jnp.max(cur, axis=0, keepdims=True)
        amax = jnp.min(jnp.where(cur == mx, e_iota, N_EXP),
                       axis=0, keepdims=True)
        idx_rows.append(amax)
        wgt_rows.append(mx)
        cur = jnp.where(e_iota == amax, -1.0, cur)
    idx_ref[...] = jnp.concatenate(idx_rows, axis=0).T
    wgt_ref[...] = jnp.concatenate(wgt_rows, axis=0).T


def _tc_router(hidden, W):
    n_tok = hidden.shape[0]
    return pl.pallas_call(
        _router_body,
        grid=(n_tok // BLK,),
        in_specs=[
            pl.BlockSpec((BLK, D_MODEL), lambda i: (i, 0)),
            pl.BlockSpec((N_EXP, D_MODEL), lambda i: (0, 0)),
        ],
        out_specs=(
            pl.BlockSpec((BLK, K_TOP), lambda i: (i, 0)),
            pl.BlockSpec((BLK, K_TOP), lambda i: (i, 0)),
            pl.BlockSpec((BLK, N_EXP), lambda i: (i, 0)),
        ),
        out_shape=(
            jax.ShapeDtypeStruct((n_tok, K_TOP), jnp.int32),
            jax.ShapeDtypeStruct((n_tok, K_TOP), jnp.float32),
            jax.ShapeDtypeStruct((n_tok, N_EXP), jnp.float32),
        ),
    )(hidden, W)


def _sc_route_body(tpw, logit_hbm, idx_hbm, wgt_hbm, lg_v, oi_v, ow_v):
    wid = lax.axis_index("s") * 2 + lax.axis_index("c")
    base = wid * tpw
    pltpu.sync_copy(logit_hbm.at[pl.ds(base * N_EXP, tpw * N_EXP)], lg_v)

    lane = lax.iota(jnp.int32, LANES)
    lane_e = lane * N_EXP
    lane_k = lane * K_TOP

    def group(g, _):
        gbase_e = g * (LANES * N_EXP)
        gbase_k = g * (LANES * K_TOP)
        neg_inf = jnp.full((LANES,), -jnp.inf, jnp.float32)
        s = [neg_inf for _ in range(K_TOP)]
        si = [jnp.zeros((LANES,), jnp.int32) for _ in range(K_TOP)]
        m = neg_inf
        for e in range(N_EXP):
            x = plsc.load_gather(lg_v, [lane_e + (gbase_e + e)])
            m = jnp.maximum(m, x)
            xi = jnp.full((LANES,), e, jnp.int32)
            for j in range(K_TOP):
                c = x > s[j]
                nv = jnp.where(c, x, s[j])
                ni = jnp.where(c, xi, si[j])
                x = jnp.where(c, s[j], x)
                xi = jnp.where(c, si[j], xi)
                s[j], si[j] = nv, ni
        acc = jnp.zeros((LANES,), jnp.float32)
        for e in range(N_EXP):
            x = plsc.load_gather(lg_v, [lane_e + (gbase_e + e)])
            acc = acc + jnp.exp(x - m)
        for j in range(K_TOP):
            oidx = lane_k + (gbase_k + j)
            plsc.store_scatter(oi_v, [oidx], si[j])
            plsc.store_scatter(ow_v, [oidx], jnp.exp(s[j] - m) / acc)
        return _

    lax.fori_loop(0, tpw // LANES, group, None)

    pltpu.sync_copy(oi_v, idx_hbm.at[pl.ds(base * K_TOP, tpw * K_TOP)])
    pltpu.sync_copy(ow_v, wgt_hbm.at[pl.ds(base * K_TOP, tpw * K_TOP)])


def _make_sc_route(sc_tokens):
    tpw = sc_tokens // N_WORKERS
    mesh = plsc.VectorSubcoreMesh(core_axis_name="c", subcore_axis_name="s")
    return pl.kernel(
        functools.partial(_sc_route_body, tpw),
        mesh=mesh,
        out_type=[
            jax.ShapeDtypeStruct((sc_tokens * K_TOP,), jnp.int32),
            jax.ShapeDtypeStruct((sc_tokens * K_TOP,), jnp.float32),
        ],
        scratch_types=[
            pltpu.VMEM((tpw * N_EXP,), jnp.float32),
            pltpu.VMEM((tpw * K_TOP,), jnp.int32),
            pltpu.VMEM((tpw * K_TOP,), jnp.float32),
        ],
        compiler_params=pltpu.CompilerParams(needs_layout_passes=False),
    )


def kernel(layer_idx, hidden, W):
    idx_tc, wgt_tc, logits = _tc_router(hidden, W)
    sc_idx, sc_wgt = _make_sc_route(SC_TOKENS)(logits.reshape(-1))
    idx = lax.dynamic_update_slice(
        idx_tc, sc_idx.reshape(SC_TOKENS, K_TOP), (0, 0))
    wgt = lax.dynamic_update_slice(
        wgt_tc, sc_wgt.reshape(SC_TOKENS, K_TOP), (0, 0))
    return (idx, wgt, logits)


# speedup vs baseline: 1.2944x; 1.0101x over previous
import functools

import jax
import jax.numpy as jnp
from jax import lax
from jax.experimental import pallas as pl
from jax.experimental.pallas import tpu as pltpu
from jax.experimental.pallas import tpu_sc as plsc

N_TOK = 32768
D_MODEL = 4096
N_EXP = 64
K_TOP = 8
BLK = 1024
LANES = 16
N_WORKERS = 32
SC_TOKENS = 2048


def _router_body(h_ref, w_ref, idx_ref, wgt_ref, logit_ref):
    lt = jax.lax.dot_general(
        w_ref[...], h_ref[...], (((1,), (1,)), ((), ())),
        preferred_element_type=jnp.float32)
    logit_ref[...] = lt.T

    m = jnp.max(lt, axis=0, keepdims=True)
    ex = jnp.exp(lt - m)
    probs = ex / jnp.sum(ex, axis=0, keepdims=True)

    cur = probs
    e_iota = jax.lax.broadcasted_iota(jnp.int32, cur.shape, 0)
    idx_rows = []
    wgt_rows = []
    for _ in range(K_TOP):
        mx = jnp.max(cur, axis=0, keepdims=True)
        amax = jnp.min(jnp.where(cur == mx, e_iota, N_EXP),
                       axis=0, keepdims=True)
        idx_rows.append(amax)
        wgt_rows.append(mx)
        cur = jnp.where(e_iota == amax, -1.0, cur)
    idx_ref[...] = jnp.concatenate(idx_rows, axis=0).T
    wgt_ref[...] = jnp.concatenate(wgt_rows, axis=0).T


def _tc_router(hidden, W):
    n_tok = hidden.shape[0]
    return pl.pallas_call(
        _router_body,
        grid=(n_tok // BLK,),
        in_specs=[
            pl.BlockSpec((BLK, D_MODEL), lambda i: (i, 0)),
            pl.BlockSpec((N_EXP, D_MODEL), lambda i: (0, 0)),
        ],
        out_specs=(
            pl.BlockSpec((BLK, K_TOP), lambda i: (i, 0)),
            pl.BlockSpec((BLK, K_TOP), lambda i: (i, 0)),
            pl.BlockSpec((BLK, N_EXP), lambda i: (i, 0)),
        ),
        out_shape=(
            jax.ShapeDtypeStruct((n_tok, K_TOP), jnp.int32),
            jax.ShapeDtypeStruct((n_tok, K_TOP), jnp.float32),
            jax.ShapeDtypeStruct((n_tok, N_EXP), jnp.float32),
        ),
    )(hidden, W)


def _sc_route_body(tpw, logit_hbm, idx_hbm, wgt_hbm, lg_v, oi_v, ow_v):
    wid = lax.axis_index("s") * 2 + lax.axis_index("c")
    base = wid * tpw
    pltpu.sync_copy(logit_hbm.at[pl.ds(base * N_EXP, tpw * N_EXP)], lg_v)

    lane = lax.iota(jnp.int32, LANES)
    lane_e = lane * N_EXP
    lane_k = lane * K_TOP

    def group(g, _):
        gbase_e = g * (LANES * N_EXP)
        gbase_k = g * (LANES * K_TOP)
        neg_inf = jnp.full((LANES,), -jnp.inf, jnp.float32)
        s = [neg_inf for _ in range(K_TOP)]
        si = [jnp.zeros((LANES,), jnp.int32) for _ in range(K_TOP)]
        m = neg_inf
        for e in range(N_EXP):
            x = plsc.load_gather(lg_v, [lane_e + (gbase_e + e)])
            m = jnp.maximum(m, x)
            xi = jnp.full((LANES,), e, jnp.int32)
            for j in range(K_TOP):
                c = x > s[j]
                nv = jnp.where(c, x, s[j])
                ni = jnp.where(c, xi, si[j])
                x = jnp.where(c, s[j], x)
                xi = jnp.where(c, si[j], xi)
                s[j], si[j] = nv, ni
        acc = jnp.zeros((LANES,), jnp.float32)
        for e in range(N_EXP):
            x = plsc.load_gather(lg_v, [lane_e + (gbase_e + e)])
            acc = acc + jnp.exp(x - m)
        for j in range(K_TOP):
            oidx = lane_k + (gbase_k + j)
            plsc.store_scatter(oi_v, [oidx], si[j])
            plsc.store_scatter(ow_v, [oidx], jnp.exp(s[j] - m) / acc)
        return _

    lax.fori_loop(0, tpw // LANES, group, None)

    pltpu.sync_copy(oi_v, idx_hbm.at[pl.ds(base * K_TOP, tpw * K_TOP)])
    pltpu.sync_copy(ow_v, wgt_hbm.at[pl.ds(base * K_TOP, tpw * K_TOP)])


def _make_sc_route(sc_tokens):
    tpw = sc_tokens // N_WORKERS
    mesh = plsc.VectorSubcoreMesh(core_axis_name="c", subcore_axis_name="s")
    return pl.kernel(
        functools.partial(_sc_route_body, tpw),
        mesh=mesh,
        out_type=[
            jax.ShapeDtypeStruct((sc_tokens * K_TOP,), jnp.int32),
            jax.ShapeDtypeStruct((sc_tokens * K_TOP,), jnp.float32),
        ],
        scratch_types=[
            pltpu.VMEM((tpw * N_EXP,), jnp.float32),
            pltpu.VMEM((tpw * K_TOP,), jnp.int32),
            pltpu.VMEM((tpw * K_TOP,), jnp.float32),
        ],
        compiler_params=pltpu.CompilerParams(needs_layout_passes=False),
    )


def kernel(layer_idx, hidden, W):
    idx_tc, wgt_tc, logits = _tc_router(hidden, W)
    sc_idx, sc_wgt = _make_sc_route(SC_TOKENS)(logits.reshape(-1))
    idx = lax.dynamic_update_slice(
        idx_tc, sc_idx.reshape(SC_TOKENS, K_TOP), (0, 0))
    wgt = lax.dynamic_update_slice(
        wgt_tc, sc_wgt.reshape(SC_TOKENS, K_TOP), (0, 0))
    return (idx, wgt, logits)


# speedup vs baseline: 1.3517x; 1.0443x over previous
import functools

import jax
import jax.numpy as jnp
from jax import lax
from jax.experimental import pallas as pl
from jax.experimental.pallas import tpu as pltpu
from jax.experimental.pallas import tpu_sc as plsc

N_TOK = 32768
D_MODEL = 4096
N_EXP = 64
K_TOP = 8
BLK = 1024
LANES = 16
N_WORKERS = 32
SC_TOKENS = 2048


def _router_body(h_ref, w_ref, idx_ref, wgt_ref, logit_ref):
    lt = jax.lax.dot_general(
        w_ref[...], h_ref[...], (((1,), (1,)), ((), ())),
        preferred_element_type=jnp.float32)
    logit_ref[...] = lt.T

    m = jnp.max(lt, axis=0, keepdims=True)
    ex = jnp.exp(lt - m)
    probs = ex / jnp.sum(ex, axis=0, keepdims=True)

    cur = probs
    e_iota = jax.lax.broadcasted_iota(jnp.int32, cur.shape, 0)
    idx_rows = []
    wgt_rows = []
    for _ in range(K_TOP):
        mx = jnp.max(cur, axis=0, keepdims=True)
        amax = jnp.min(jnp.where(cur == mx, e_iota, N_EXP),
                       axis=0, keepdims=True)
        idx_rows.append(amax)
        wgt_rows.append(mx)
        cur = jnp.where(e_iota == amax, -1.0, cur)
    idx_ref[...] = jnp.concatenate(idx_rows, axis=0).T
    wgt_ref[...] = jnp.concatenate(wgt_rows, axis=0).T


def _logits_body(h_ref, w_ref, logit_ref):
    lt = jax.lax.dot_general(
        w_ref[...], h_ref[...], (((1,), (1,)), ((), ())),
        preferred_element_type=jnp.float32)
    logit_ref[...] = lt.T


def _tc_router(hidden, W, tok0, n_tok):
    blk0 = tok0 // BLK
    return pl.pallas_call(
        _router_body,
        grid=(n_tok // BLK,),
        in_specs=[
            pl.BlockSpec((BLK, D_MODEL), lambda i: (i + blk0, 0)),
            pl.BlockSpec((N_EXP, D_MODEL), lambda i: (0, 0)),
        ],
        out_specs=(
            pl.BlockSpec((BLK, K_TOP), lambda i: (i, 0)),
            pl.BlockSpec((BLK, K_TOP), lambda i: (i, 0)),
            pl.BlockSpec((BLK, N_EXP), lambda i: (i, 0)),
        ),
        out_shape=(
            jax.ShapeDtypeStruct((n_tok, K_TOP), jnp.int32),
            jax.ShapeDtypeStruct((n_tok, K_TOP), jnp.float32),
            jax.ShapeDtypeStruct((n_tok, N_EXP), jnp.float32),
        ),
    )(hidden, W)


def _tc_logits(hidden, W, tok0, n_tok):
    blk0 = tok0 // BLK
    return pl.pallas_call(
        _logits_body,
        grid=(n_tok // BLK,),
        in_specs=[
            pl.BlockSpec((BLK, D_MODEL), lambda i: (i + blk0, 0)),
            pl.BlockSpec((N_EXP, D_MODEL), lambda i: (0, 0)),
        ],
        out_specs=pl.BlockSpec((BLK, N_EXP), lambda i: (i, 0)),
        out_shape=jax.ShapeDtypeStruct((n_tok, N_EXP), jnp.float32),
    )(hidden, W)


def _sc_route_body(tpw, logit_hbm, idx_hbm, wgt_hbm, lg_v, oi_v, ow_v):
    wid = lax.axis_index("s") * 2 + lax.axis_index("c")
    base = wid * tpw
    pltpu.sync_copy(logit_hbm.at[pl.ds(base * N_EXP, tpw * N_EXP)], lg_v)

    lane = lax.iota(jnp.int32, LANES)
    lane_e = lane * N_EXP
    lane_k = lane * K_TOP

    def group(g, _):
        gbase_e = g * (LANES * N_EXP)
        gbase_k = g * (LANES * K_TOP)
        neg_inf = jnp.full((LANES,), -jnp.inf, jnp.float32)
        s = [neg_inf for _ in range(K_TOP)]
        si = [jnp.zeros((LANES,), jnp.int32) for _ in range(K_TOP)]
        m = neg_inf
        for e in range(N_EXP):
            x = plsc.load_gather(lg_v, [lane_e + (gbase_e + e)])
            m = jnp.maximum(m, x)
            xi = jnp.full((LANES,), e, jnp.int32)
            for j in range(K_TOP):
                c = x > s[j]
                nv = jnp.where(c, x, s[j])
                ni = jnp.where(c, xi, si[j])
                x = jnp.where(c, s[j], x)
                xi = jnp.where(c, si[j], xi)
                s[j], si[j] = nv, ni
        acc = jnp.zeros((LANES,), jnp.float32)
        for e in range(N_EXP):
            x = plsc.load_gather(lg_v, [lane_e + (gbase_e + e)])
            acc = acc + jnp.exp(x - m)
        for j in range(K_TOP):
            oidx = lane_k + (gbase_k + j)
            plsc.store_scatter(oi_v, [oidx], si[j])
            plsc.store_scatter(ow_v, [oidx], jnp.exp(s[j] - m) / acc)
        return _

    lax.fori_loop(0, tpw // LANES, group, None)

    pltpu.sync_copy(oi_v, idx_hbm.at[pl.ds(base * K_TOP, tpw * K_TOP)])
    pltpu.sync_copy(ow_v, wgt_hbm.at[pl.ds(base * K_TOP, tpw * K_TOP)])


def _make_sc_route(sc_tokens):
    tpw = sc_tokens // N_WORKERS
    mesh = plsc.VectorSubcoreMesh(core_axis_name="c", subcore_axis_name="s")
    return pl.kernel(
        functools.partial(_sc_route_body, tpw),
        mesh=mesh,
        out_type=[
            jax.ShapeDtypeStruct((sc_tokens * K_TOP,), jnp.int32),
            jax.ShapeDtypeStruct((sc_tokens * K_TOP,), jnp.float32),
        ],
        scratch_types=[
            pltpu.VMEM((tpw * N_EXP,), jnp.float32),
            pltpu.VMEM((tpw * K_TOP,), jnp.int32),
            pltpu.VMEM((tpw * K_TOP,), jnp.float32),
        ],
        compiler_params=pltpu.CompilerParams(needs_layout_passes=False),
    )


def kernel(layer_idx, hidden, W):
    n_tok = hidden.shape[0]
    logits_head = _tc_logits(hidden, W, 0, SC_TOKENS)
    sc_idx, sc_wgt = _make_sc_route(SC_TOKENS)(logits_head.reshape(-1))
    idx_tail, wgt_tail, logits_tail = _tc_router(
        hidden, W, SC_TOKENS, n_tok - SC_TOKENS)
    idx = jnp.concatenate([sc_idx.reshape(SC_TOKENS, K_TOP), idx_tail], 0)
    wgt = jnp.concatenate([sc_wgt.reshape(SC_TOKENS, K_TOP), wgt_tail], 0)
    logits = jnp.concatenate([logits_head, logits_tail], 0)
    return (idx, wgt, logits)


# speedup vs baseline: 1.3765x; 1.0184x over previous
import functools

import jax
import jax.numpy as jnp
from jax import lax
from jax.experimental import pallas as pl
from jax.experimental.pallas import tpu as pltpu
from jax.experimental.pallas import tpu_sc as plsc

N_TOK = 32768
D_MODEL = 4096
N_EXP = 64
K_TOP = 8
BLK = 1024
LANES = 16
N_WORKERS = 32
SC_TOKENS = 2048


def _router_body(h_ref, w_ref, idx_ref, wgt_ref, logit_ref):
    lt = jax.lax.dot_general(
        w_ref[...], h_ref[...], (((1,), (1,)), ((), ())),
        preferred_element_type=jnp.float32)
    logit_ref[...] = lt.T

    m = jnp.max(lt, axis=0, keepdims=True)
    ex = jnp.exp(lt - m)
    probs = ex / jnp.sum(ex, axis=0, keepdims=True)

    cur = probs
    e_iota = jax.lax.broadcasted_iota(jnp.int32, cur.shape, 0)
    idx_rows = []
    wgt_rows = []
    for _ in range(K_TOP):
        mx = jnp.max(cur, axis=0, keepdims=True)
        amax = jnp.min(jnp.where(cur == mx, e_iota, N_EXP),
                       axis=0, keepdims=True)
        idx_rows.append(amax)
        wgt_rows.append(mx)
        cur = jnp.where(e_iota == amax, -1.0, cur)
    idx_ref[...] = jnp.concatenate(idx_rows, axis=0).T
    wgt_ref[...] = jnp.concatenate(wgt_rows, axis=0).T


def _logits_body(h_ref, w_ref, logit_ref):
    lt = jax.lax.dot_general(
        w_ref[...], h_ref[...], (((1,), (1,)), ((), ())),
        preferred_element_type=jnp.float32)
    logit_ref[...] = lt.T


def _tc_router(hidden, W, tok0):
    n_tok = hidden.shape[0]
    blk0 = tok0 // BLK
    return pl.pallas_call(
        _router_body,
        grid=(n_tok // BLK - blk0,),
        in_specs=[
            pl.BlockSpec((BLK, D_MODEL), lambda i: (i + blk0, 0)),
            pl.BlockSpec((N_EXP, D_MODEL), lambda i: (0, 0)),
        ],
        out_specs=(
            pl.BlockSpec((BLK, K_TOP), lambda i: (i + blk0, 0)),
            pl.BlockSpec((BLK, K_TOP), lambda i: (i + blk0, 0)),
            pl.BlockSpec((BLK, N_EXP), lambda i: (i + blk0, 0)),
        ),
        out_shape=(
            jax.ShapeDtypeStruct((n_tok, K_TOP), jnp.int32),
            jax.ShapeDtypeStruct((n_tok, K_TOP), jnp.float32),
            jax.ShapeDtypeStruct((n_tok, N_EXP), jnp.float32),
        ),
    )(hidden, W)


def _tc_logits(hidden, W, tok0, n_tok):
    blk0 = tok0 // BLK
    return pl.pallas_call(
        _logits_body,
        grid=(n_tok // BLK,),
        in_specs=[
            pl.BlockSpec((BLK, D_MODEL), lambda i: (i + blk0, 0)),
            pl.BlockSpec((N_EXP, D_MODEL), lambda i: (0, 0)),
        ],
        out_specs=pl.BlockSpec((BLK, N_EXP), lambda i: (i, 0)),
        out_shape=jax.ShapeDtypeStruct((n_tok, N_EXP), jnp.float32),
    )(hidden, W)


def _sc_route_body(tpw, logit_hbm, idx_hbm, wgt_hbm, lg_v, oi_v, ow_v):
    wid = lax.axis_index("s") * 2 + lax.axis_index("c")
    base = wid * tpw
    pltpu.sync_copy(logit_hbm.at[pl.ds(base * N_EXP, tpw * N_EXP)], lg_v)

    lane = lax.iota(jnp.int32, LANES)
    lane_e = lane * N_EXP
    lane_k = lane * K_TOP

    def group(g, _):
        gbase_e = g * (LANES * N_EXP)
        gbase_k = g * (LANES * K_TOP)
        neg_inf = jnp.full((LANES,), -jnp.inf, jnp.float32)
        s = [neg_inf for _ in range(K_TOP)]
        si = [jnp.zeros((LANES,), jnp.int32) for _ in range(K_TOP)]
        m = neg_inf
        for e in range(N_EXP):
            x = plsc.load_gather(lg_v, [lane_e + (gbase_e + e)])
            m = jnp.maximum(m, x)
            xi = jnp.full((LANES,), e, jnp.int32)
            for j in range(K_TOP):
                c = x > s[j]
                nv = jnp.where(c, x, s[j])
                ni = jnp.where(c, xi, si[j])
                x = jnp.where(c, s[j], x)
                xi = jnp.where(c, si[j], xi)
                s[j], si[j] = nv, ni
        acc = jnp.zeros((LANES,), jnp.float32)
        for e in range(N_EXP):
            x = plsc.load_gather(lg_v, [lane_e + (gbase_e + e)])
            acc = acc + jnp.exp(x - m)
        for j in range(K_TOP):
            oidx = lane_k + (gbase_k + j)
            plsc.store_scatter(oi_v, [oidx], si[j])
            plsc.store_scatter(ow_v, [oidx], jnp.exp(s[j] - m) / acc)
        return _

    lax.fori_loop(0, tpw // LANES, group, None)

    pltpu.sync_copy(oi_v, idx_hbm.at[pl.ds(base * K_TOP, tpw * K_TOP)])
    pltpu.sync_copy(ow_v, wgt_hbm.at[pl.ds(base * K_TOP, tpw * K_TOP)])


def _make_sc_route(sc_tokens):
    tpw = sc_tokens // N_WORKERS
    mesh = plsc.VectorSubcoreMesh(core_axis_name="c", subcore_axis_name="s")
    return pl.kernel(
        functools.partial(_sc_route_body, tpw),
        mesh=mesh,
        out_type=[
            jax.ShapeDtypeStruct((sc_tokens * K_TOP,), jnp.int32),
            jax.ShapeDtypeStruct((sc_tokens * K_TOP,), jnp.float32),
        ],
        scratch_types=[
            pltpu.VMEM((tpw * N_EXP,), jnp.float32),
            pltpu.VMEM((tpw * K_TOP,), jnp.int32),
            pltpu.VMEM((tpw * K_TOP,), jnp.float32),
        ],
        compiler_params=pltpu.CompilerParams(needs_layout_passes=False),
    )


def kernel(layer_idx, hidden, W):
    logits_head = _tc_logits(hidden, W, 0, SC_TOKENS)
    sc_idx, sc_wgt = _make_sc_route(SC_TOKENS)(logits_head.reshape(-1))
    idx_f, wgt_f, logits_f = _tc_router(hidden, W, SC_TOKENS)
    idx = lax.dynamic_update_slice(
        idx_f, sc_idx.reshape(SC_TOKENS, K_TOP), (0, 0))
    wgt = lax.dynamic_update_slice(
        wgt_f, sc_wgt.reshape(SC_TOKENS, K_TOP), (0, 0))
    logits = lax.dynamic_update_slice(logits_f, logits_head, (0, 0))
    return (idx, wgt, logits)
